# baseline (speedup 1.0000x reference)
import functools

import jax
import jax.numpy as jnp
from jax import lax
from jax.experimental import pallas as pl
from jax.experimental.pallas import tpu as pltpu
from jax.experimental.pallas import tpu_sc as plsc

NC = 2
NS = 16
L = 16
NW = NC * NS

_mesh = functools.partial(
    plsc.VectorSubcoreMesh, core_axis_name="c", subcore_axis_name="s")

_sc_params = pltpu.CompilerParams(needs_layout_passes=False)


def _emb_gather(emb, idx_pad):
    BP = idx_pad.shape[0]
    D = emb.shape[1]
    bpw = BP // NW
    bb = 80
    nb = bpw // bb

    @functools.partial(
        pl.kernel,
        mesh=_mesh(),
        out_type=jax.ShapeDtypeStruct((BP, D), jnp.float32),
        compiler_params=_sc_params,
        scratch_types=[
            pltpu.VMEM((bb,), jnp.int32),
            pltpu.VMEM((bb, D), jnp.float32),
            pltpu.SemaphoreType.DMA,
        ],
    )
    def k(emb_hbm, idx_hbm, out_hbm, idx_v, rows_v, sem):
        wid = lax.axis_index("s") * NC + lax.axis_index("c")

        def body(j, carry):
            base = wid * bpw + j * bb
            pltpu.sync_copy(idx_hbm.at[pl.ds(base, bb)], idx_v)
            pltpu.async_copy(emb_hbm.at[idx_v], rows_v, sem).wait()
            pltpu.sync_copy(rows_v, out_hbm.at[pl.ds(base, bb)])
            return carry

        lax.fori_loop(0, nb, body, 0)

    return k(emb, idx_pad)


def _degree(dst, w, np_rows):
    E = dst.shape[0]
    epw = ((E // NW) + 15) // 16 * 16
    last = E - (NW - 1) * epw

    @functools.partial(
        pl.kernel,
        mesh=_mesh(),
        out_type=jax.ShapeDtypeStruct((NW, np_rows), jnp.float32),
        compiler_params=_sc_params,
        scratch_types=[
            pltpu.VMEM((128,), jnp.int32),
            pltpu.VMEM((128,), jnp.float32),
            pltpu.VMEM((16,), jnp.int32),
            pltpu.VMEM((16,), jnp.float32),
            pltpu.VMEM((np_rows,), jnp.float32),
        ],
    )
    def k(dst_hbm, w_hbm, out_hbm, didx, wv, didx_t, wv_t, deg_l):
        wid = lax.axis_index("s") * NC + lax.axis_index("c")
        z16 = jnp.zeros((L,), jnp.float32)

        def zb(i, carry):
            deg_l[pl.ds(i * L, L)] = z16
            return carry

        lax.fori_loop(0, np_rows // L, zb, 0)

        base = wid * epw
        cnt = jnp.where(wid == NW - 1, last, epw)
        nbatch = (cnt - 16) // 128

        def bb_(j, carry):
            eb = base + j * 128
            pltpu.sync_copy(dst_hbm.at[pl.ds(eb, 128)], didx)
            pltpu.sync_copy(w_hbm.at[pl.ds(eb, 128)], wv)
            for t in range(8):
                sl = pl.ds(t * L, L)
                plsc.addupdate_scatter(deg_l, (didx[sl],), wv[sl])
            return carry

        lax.fori_loop(0, nbatch, bb_, 0)

        def tb(j, carry):
            eb = base + nbatch * 128 + j * 16
            pltpu.sync_copy(dst_hbm.at[pl.ds(eb, 16)], didx_t)
            pltpu.sync_copy(w_hbm.at[pl.ds(eb, 16)], wv_t)
            plsc.addupdate_scatter(deg_l, (didx_t[...],), wv_t[...])
            return carry

        lax.fori_loop(0, (cnt - nbatch * 128) // 16, tb, 0)

        pltpu.sync_copy(deg_l, out_hbm.at[wid])

    return k(dst, w)


def _spmm(xws_flat, src, dst, w, np_rows, n_rows):
    E = src.shape[0]
    F = xws_flat.shape[1]
    eps = E // NS
    BB = 80
    nbatch = eps // BB
    ntri = (nbatch - 3) // 3
    rem = (nbatch - 3) - 3 * ntri
    assert eps * NS == E and nbatch * BB == eps and nbatch >= 3
    rpw = np_rows // NS
    assert rpw % BB == 0

    @functools.partial(
        pl.kernel,
        mesh=_mesh(),
        out_type=jax.ShapeDtypeStruct((NC, np_rows, F), jnp.float32),
        compiler_params=_sc_params,
        scratch_types=[
            pltpu.VMEM((BB,), jnp.int32),
            pltpu.VMEM((BB,), jnp.int32),
            pltpu.VMEM((BB,), jnp.float32),
            pltpu.VMEM((BB, F), jnp.float32),
            pltpu.VMEM((BB,), jnp.int32),
            pltpu.VMEM((BB,), jnp.int32),
            pltpu.VMEM((BB,), jnp.float32),
            pltpu.VMEM((BB, F), jnp.float32),
            pltpu.VMEM((BB,), jnp.int32),
            pltpu.VMEM((BB,), jnp.int32),
            pltpu.VMEM((BB,), jnp.float32),
            pltpu.VMEM((BB, F), jnp.float32),
            pltpu.VMEM_SHARED((np_rows, F), jnp.float32),
            pltpu.SemaphoreType.DMA,
            pltpu.SemaphoreType.DMA,
            pltpu.SemaphoreType.DMA,
            pltpu.SemaphoreType.DMA,
            pltpu.SemaphoreType.DMA,
            pltpu.SemaphoreType.DMA,
        ],
    )
    def k(xws_hbm, src_hbm, dst_hbm, w_hbm, out_hbm,
          sidx0, didx0, wv0, rows0, sidx1, didx1, wv1, rows1,
          sidx2, didx2, wv2, rows2,
          acc_sh, gsem0, gsem1, gsem2, ssem0, ssem1, ssem2):
        c = lax.axis_index("c")
        s = lax.axis_index("s")
        nvr = F // L
        z16 = jnp.zeros((L,), jnp.float32)
        bufs = ((sidx0, didx0, wv0, rows0, gsem0, ssem0),
                (sidx1, didx1, wv1, rows1, gsem1, ssem1),
                (sidx2, didx2, wv2, rows2, gsem2, ssem2))

        @plsc.parallel_loop(0, BB, unroll=4)
        def _(i):
            for t in range(nvr):
                rows0[i, pl.ds(t * L, L)] = z16

        def zc(t, carry):
            pltpu.sync_copy(rows0, acc_sh.at[pl.ds(s * rpw + t * BB, BB)])
            return carry

        lax.fori_loop(0, rpw // BB, zc, 0)
        plsc.subcore_barrier()

        coff = c * n_rows
        base = s * eps

        def scale(rows_ref, wv_ref, n):
            @plsc.parallel_loop(0, n, unroll=4)
            def _(e):
                we = plsc.load_gather(wv_ref, (jnp.full((L,), e, jnp.int32),))
                for t in range(nvr):
                    sl = pl.ds(t * L, L)
                    rows_ref[e, sl] = rows_ref[e, sl] * we

        def start_gather(j, b):
            sidx, didx, wv, rows, gsem, _ = bufs[b]
            eb = base + j * BB
            pltpu.sync_copy(src_hbm.at[pl.ds(eb, BB)], sidx)
            pltpu.sync_copy(dst_hbm.at[pl.ds(eb, BB)], didx)
            pltpu.sync_copy(w_hbm.at[pl.ds(eb, BB)], wv)
            for t in range(BB // L):
                sl = pl.ds(t * L, L)
                sidx[sl] = sidx[sl] + coff
            pltpu.async_copy(xws_hbm.at[sidx], rows, gsem)

        def finish(b):
            sidx, didx, wv, rows, gsem, ssem = bufs[b]
            pltpu.make_async_copy(xws_hbm.at[sidx], rows, gsem).wait()
            scale(rows, wv, BB)
            pltpu.async_copy(rows, acc_sh.at[didx], ssem, add=True)

        def wait_scatter(b):
            _, didx, _, rows, _, ssem = bufs[b]
            pltpu.make_async_copy(rows, acc_sh.at[didx], ssem).wait()

        start_gather(0, 0)
        start_gather(1, 1)
        start_gather(2, 2)
        finish(0)

        def tri(t, carry):
            for p in range(3):
                j = 3 * t + 3 + p
                wait_scatter(p)
                start_gather(j, p)
                finish((p + 1) % 3)
            return carry

        lax.fori_loop(0, ntri, tri, 0)

        for p in range(rem):
            j = 3 * ntri + 3 + p
            wait_scatter(j % 3)
            start_gather(j, j % 3)
            finish((j + 1) % 3)

        finish((nbatch - 2) % 3)
        finish((nbatch - 1) % 3)
        wait_scatter((nbatch - 3) % 3)
        wait_scatter((nbatch - 2) % 3)
        wait_scatter((nbatch - 1) % 3)

        plsc.subcore_barrier()

        def co(t, carry):
            r0 = s * rpw + t * BB
            pltpu.sync_copy(acc_sh.at[pl.ds(r0, BB)], rows0)
            pltpu.sync_copy(rows0, out_hbm.at[c, pl.ds(r0, BB)])
            return carry

        lax.fori_loop(0, rpw // BB, co, 0)

    return k(xws_flat, src, dst, w)


def _dis_of(deg_ref):
    deg = jnp.sum(deg_ref[...], axis=1, keepdims=True) + 1.0
    return jnp.where(deg > 0, lax.rsqrt(jnp.maximum(deg, 1e-12)), 0.0)


def _row_specs(nblk, rows, cols_list):
    return [pl.BlockSpec((rows, c), lambda i: (i, 0)) for c in cols_list]


def _tc_pre(x, W, deg_t):
    n, d = x.shape
    rows = n // 10

    def body(x_ref, w_ref, deg_ref, xw_ref, xsa_ref, xsb_ref):
        dis = _dis_of(deg_ref)
        xw = jnp.dot(x_ref[...], w_ref[...], preferred_element_type=jnp.float32)
        xw_ref[...] = xw
        xs = xw * dis
        xsa_ref[...] = xs[:, : d // 2]
        xsb_ref[...] = xs[:, d // 2:]

    return pl.pallas_call(
        body,
        grid=(10,),
        in_specs=[
            pl.BlockSpec((rows, d), lambda i: (i, 0)),
            pl.BlockSpec((d, d), lambda i: (0, 0)),
            pl.BlockSpec((rows, NW), lambda i: (i, 0)),
        ],
        out_specs=[
            pl.BlockSpec((rows, d), lambda i: (i, 0)),
            pl.BlockSpec((rows, d // 2), lambda i: (i, 0)),
            pl.BlockSpec((rows, d // 2), lambda i: (i, 0)),
        ],
        out_shape=[
            jax.ShapeDtypeStruct((n, d), jnp.float32),
            jax.ShapeDtypeStruct((n, d // 2), jnp.float32),
            jax.ShapeDtypeStruct((n, d // 2), jnp.float32),
        ],
    )(x, W, deg_t)


def _tc_mid(acc, xw_prev, deg_t, b_prev, W_next):
    n, d = acc.shape
    rows = n // 10

    def body(acc_ref, xw_ref, deg_ref, b_ref, w_ref, xwn_ref, xsa_ref, xsb_ref):
        dis = _dis_of(deg_ref)
        h = dis * acc_ref[...] + (dis * dis) * xw_ref[...] + b_ref[...]
        xwn = jnp.dot(h, w_ref[...], preferred_element_type=jnp.float32)
        xwn_ref[...] = xwn
        xs = xwn * dis
        xsa_ref[...] = xs[:, : d // 2]
        xsb_ref[...] = xs[:, d // 2:]

    return pl.pallas_call(
        body,
        grid=(10,),
        in_specs=[
            pl.BlockSpec((rows, d), lambda i: (i, 0)),
            pl.BlockSpec((rows, d), lambda i: (i, 0)),
            pl.BlockSpec((rows, NW), lambda i: (i, 0)),
            pl.BlockSpec((1, d), lambda i: (0, 0)),
            pl.BlockSpec((d, d), lambda i: (0, 0)),
        ],
        out_specs=[
            pl.BlockSpec((rows, d), lambda i: (i, 0)),
            pl.BlockSpec((rows, d // 2), lambda i: (i, 0)),
            pl.BlockSpec((rows, d // 2), lambda i: (i, 0)),
        ],
        out_shape=[
            jax.ShapeDtypeStruct((n, d), jnp.float32),
            jax.ShapeDtypeStruct((n, d // 2), jnp.float32),
            jax.ShapeDtypeStruct((n, d // 2), jnp.float32),
        ],
    )(acc, xw_prev, deg_t, b_prev, W_next)


def _tc_post(acc, xw, deg_t, b):
    n, d = acc.shape
    rows = n // 10

    def body(acc_ref, xw_ref, deg_ref, b_ref, out_ref):
        dis = _dis_of(deg_ref)
        out_ref[...] = dis * acc_ref[...] + (dis * dis) * xw_ref[...] + b_ref[...]

    return pl.pallas_call(
        body,
        grid=(10,),
        in_specs=[
            pl.BlockSpec((rows, d), lambda i: (i, 0)),
            pl.BlockSpec((rows, d), lambda i: (i, 0)),
            pl.BlockSpec((rows, NW), lambda i: (i, 0)),
            pl.BlockSpec((1, d), lambda i: (0, 0)),
        ],
        out_specs=pl.BlockSpec((rows, d), lambda i: (i, 0)),
        out_shape=jax.ShapeDtypeStruct((n, d), jnp.float32),
    )(acc, xw, deg_t, b)


def kernel(node_features, edge_indices, edge_weights, emb, W1, b1, W2, b2, W3, b3):
    G, n = node_features.shape
    d = emb.shape[1]
    np_rows = ((n + 8 * NW - 1) // (8 * NW)) * (8 * NW)

    b1r = b1.reshape(1, d)
    b2r = b2.reshape(1, d)
    b3r = b3.reshape(1, d)

    outs = []
    for g in range(G):
        nf = node_features[g]
        src = edge_indices[g, 0]
        dst = edge_indices[g, 1]
        w = edge_weights[g]

        deg32 = _degree(dst, w, np_rows)
        deg_t = deg32[:, :n].T

        x0 = _emb_gather(emb, jnp.pad(nf, (0, np_rows - n)))[:n]

        xw, xsa, xsb = _tc_pre(x0, W1, deg_t)
        for (b_r, W_next) in ((b1r, W2), (b2r, W3)):
            xws_flat = jnp.concatenate([xsa, xsb], axis=0)
            acc4 = _spmm(xws_flat, src, dst, w, np_rows, n)
            acc = jnp.concatenate([acc4[0, :n], acc4[1, :n]], axis=1)
            xw, xsa, xsb = _tc_mid(acc, xw, deg_t, b_r, W_next)

        xws_flat = jnp.concatenate([xsa, xsb], axis=0)
        acc4 = _spmm(xws_flat, src, dst, w, np_rows, n)
        acc = jnp.concatenate([acc4[0, :n], acc4[1, :n]], axis=1)
        outs.append(_tc_post(acc, xw, deg_t, b3r))

    return jnp.concatenate(outs, axis=0)


# speedup vs baseline: 1.1729x; 1.1729x over previous
import functools

import jax
import jax.numpy as jnp
from jax import lax
from jax.experimental import pallas as pl
from jax.experimental.pallas import tpu as pltpu
from jax.experimental.pallas import tpu_sc as plsc

NC = 2
NS = 16
L = 16
NW = NC * NS

_mesh = functools.partial(
    plsc.VectorSubcoreMesh, core_axis_name="c", subcore_axis_name="s")

_sc_params = pltpu.CompilerParams(needs_layout_passes=False)


def _emb_gather(emb, idx_pad):
    BP = idx_pad.shape[0]
    D = emb.shape[1]
    bpw = BP // NW
    bb = 80
    nb = bpw // bb

    @functools.partial(
        pl.kernel,
        mesh=_mesh(),
        out_type=jax.ShapeDtypeStruct((BP, D), jnp.float32),
        compiler_params=_sc_params,
        scratch_types=[
            pltpu.VMEM((bb,), jnp.int32),
            pltpu.VMEM((bb, D), jnp.float32),
            pltpu.SemaphoreType.DMA,
        ],
    )
    def k(emb_hbm, idx_hbm, out_hbm, idx_v, rows_v, sem):
        wid = lax.axis_index("s") * NC + lax.axis_index("c")

        def body(j, carry):
            base = wid * bpw + j * bb
            pltpu.sync_copy(idx_hbm.at[pl.ds(base, bb)], idx_v)
            pltpu.async_copy(emb_hbm.at[idx_v], rows_v, sem).wait()
            pltpu.sync_copy(rows_v, out_hbm.at[pl.ds(base, bb)])
            return carry

        lax.fori_loop(0, nb, body, 0)

    return k(emb, idx_pad)


def _degree(dst, w, np_rows):
    E = dst.shape[0]
    epw = ((E // NW) + 15) // 16 * 16
    last = E - (NW - 1) * epw

    @functools.partial(
        pl.kernel,
        mesh=_mesh(),
        out_type=jax.ShapeDtypeStruct((NW, np_rows), jnp.float32),
        compiler_params=_sc_params,
        scratch_types=[
            pltpu.VMEM((128,), jnp.int32),
            pltpu.VMEM((128,), jnp.float32),
            pltpu.VMEM((16,), jnp.int32),
            pltpu.VMEM((16,), jnp.float32),
            pltpu.VMEM((np_rows,), jnp.float32),
        ],
    )
    def k(dst_hbm, w_hbm, out_hbm, didx, wv, didx_t, wv_t, deg_l):
        wid = lax.axis_index("s") * NC + lax.axis_index("c")
        z16 = jnp.zeros((L,), jnp.float32)

        def zb(i, carry):
            deg_l[pl.ds(i * L, L)] = z16
            return carry

        lax.fori_loop(0, np_rows // L, zb, 0)

        base = wid * epw
        cnt = jnp.where(wid == NW - 1, last, epw)
        nbatch = (cnt - 16) // 128

        def bb_(j, carry):
            eb = base + j * 128
            pltpu.sync_copy(dst_hbm.at[pl.ds(eb, 128)], didx)
            pltpu.sync_copy(w_hbm.at[pl.ds(eb, 128)], wv)
            for t in range(8):
                sl = pl.ds(t * L, L)
                plsc.addupdate_scatter(deg_l, (didx[sl],), wv[sl])
            return carry

        lax.fori_loop(0, nbatch, bb_, 0)

        def tb(j, carry):
            eb = base + nbatch * 128 + j * 16
            pltpu.sync_copy(dst_hbm.at[pl.ds(eb, 16)], didx_t)
            pltpu.sync_copy(w_hbm.at[pl.ds(eb, 16)], wv_t)
            plsc.addupdate_scatter(deg_l, (didx_t[...],), wv_t[...])
            return carry

        lax.fori_loop(0, (cnt - nbatch * 128) // 16, tb, 0)

        pltpu.sync_copy(deg_l, out_hbm.at[wid])

    return k(dst, w)


def _spmm(xws_flat, src, dst, w, np_rows, n_rows):
    E = src.shape[0]
    F = xws_flat.shape[1]
    eps = E // NS
    BB = 80
    nbatch = eps // BB
    ntri = (nbatch - 3) // 3
    rem = (nbatch - 3) - 3 * ntri
    assert eps * NS == E and nbatch * BB == eps and nbatch >= 3
    rpw = np_rows // NS
    assert rpw % BB == 0

    @functools.partial(
        pl.kernel,
        mesh=_mesh(),
        out_type=jax.ShapeDtypeStruct((NC, np_rows, F), jnp.float32),
        compiler_params=_sc_params,
        scratch_types=[
            pltpu.VMEM((BB,), jnp.int32),
            pltpu.VMEM((BB,), jnp.int32),
            pltpu.VMEM((BB,), jnp.float32),
            pltpu.VMEM((BB, F), jnp.float32),
            pltpu.VMEM((BB,), jnp.int32),
            pltpu.VMEM((BB,), jnp.int32),
            pltpu.VMEM((BB,), jnp.float32),
            pltpu.VMEM((BB, F), jnp.float32),
            pltpu.VMEM((BB,), jnp.int32),
            pltpu.VMEM((BB,), jnp.int32),
            pltpu.VMEM((BB,), jnp.float32),
            pltpu.VMEM((BB, F), jnp.float32),
            pltpu.VMEM_SHARED((np_rows, F), jnp.float32),
            pltpu.SemaphoreType.DMA,
            pltpu.SemaphoreType.DMA,
            pltpu.SemaphoreType.DMA,
            pltpu.SemaphoreType.DMA,
            pltpu.SemaphoreType.DMA,
            pltpu.SemaphoreType.DMA,
        ],
    )
    def k(xws_hbm, src_hbm, dst_hbm, w_hbm, out_hbm,
          sidx0, didx0, wv0, rows0, sidx1, didx1, wv1, rows1,
          sidx2, didx2, wv2, rows2,
          acc_sh, gsem0, gsem1, gsem2, ssem0, ssem1, ssem2):
        c = lax.axis_index("c")
        s = lax.axis_index("s")
        nvr = F // L
        z16 = jnp.zeros((L,), jnp.float32)
        bufs = ((sidx0, didx0, wv0, rows0, gsem0, ssem0),
                (sidx1, didx1, wv1, rows1, gsem1, ssem1),
                (sidx2, didx2, wv2, rows2, gsem2, ssem2))

        @plsc.parallel_loop(0, BB, unroll=4)
        def _(i):
            for t in range(nvr):
                rows0[i, pl.ds(t * L, L)] = z16

        def zc(t, carry):
            pltpu.sync_copy(rows0, acc_sh.at[pl.ds(s * rpw + t * BB, BB)])
            return carry

        lax.fori_loop(0, rpw // BB, zc, 0)
        plsc.subcore_barrier()

        coff = c * n_rows
        base = s * eps

        def scale(rows_ref, wv_ref, n):
            @plsc.parallel_loop(0, n, unroll=4)
            def _(e):
                we = plsc.load_gather(wv_ref, (jnp.full((L,), e, jnp.int32),))
                for t in range(nvr):
                    sl = pl.ds(t * L, L)
                    rows_ref[e, sl] = rows_ref[e, sl] * we

        def start_gather(j, b):
            sidx, didx, wv, rows, gsem, _ = bufs[b]
            eb = base + j * BB
            pltpu.sync_copy(src_hbm.at[pl.ds(eb, BB)], sidx)
            pltpu.sync_copy(dst_hbm.at[pl.ds(eb, BB)], didx)
            pltpu.sync_copy(w_hbm.at[pl.ds(eb, BB)], wv)
            for t in range(BB // L):
                sl = pl.ds(t * L, L)
                sidx[sl] = sidx[sl] + coff
            pltpu.async_copy(xws_hbm.at[sidx], rows, gsem)

        def finish(b):
            sidx, didx, wv, rows, gsem, ssem = bufs[b]
            pltpu.make_async_copy(xws_hbm.at[sidx], rows, gsem).wait()
            pltpu.async_copy(rows, acc_sh.at[didx], ssem, add=True)

        def wait_scatter(b):
            _, didx, _, rows, _, ssem = bufs[b]
            pltpu.make_async_copy(rows, acc_sh.at[didx], ssem).wait()

        start_gather(0, 0)
        start_gather(1, 1)
        start_gather(2, 2)
        finish(0)

        def tri(t, carry):
            for p in range(3):
                j = 3 * t + 3 + p
                wait_scatter(p)
                start_gather(j, p)
                finish((p + 1) % 3)
            return carry

        lax.fori_loop(0, ntri, tri, 0)

        for p in range(rem):
            j = 3 * ntri + 3 + p
            wait_scatter(j % 3)
            start_gather(j, j % 3)
            finish((j + 1) % 3)

        finish((nbatch - 2) % 3)
        finish((nbatch - 1) % 3)
        wait_scatter((nbatch - 3) % 3)
        wait_scatter((nbatch - 2) % 3)
        wait_scatter((nbatch - 1) % 3)

        plsc.subcore_barrier()

        def co(t, carry):
            r0 = s * rpw + t * BB
            pltpu.sync_copy(acc_sh.at[pl.ds(r0, BB)], rows0)
            pltpu.sync_copy(rows0, out_hbm.at[c, pl.ds(r0, BB)])
            return carry

        lax.fori_loop(0, rpw // BB, co, 0)

    return k(xws_flat, src, dst, w)


def _dis_of(deg_ref):
    deg = jnp.sum(deg_ref[...], axis=1, keepdims=True) + 1.0
    return jnp.where(deg > 0, lax.rsqrt(jnp.maximum(deg, 1e-12)), 0.0)


def _row_specs(nblk, rows, cols_list):
    return [pl.BlockSpec((rows, c), lambda i: (i, 0)) for c in cols_list]


def _tc_pre(x, W, deg_t):
    n, d = x.shape
    rows = n // 10

    def body(x_ref, w_ref, deg_ref, xw_ref, xsa_ref, xsb_ref):
        dis = _dis_of(deg_ref)
        xw = jnp.dot(x_ref[...], w_ref[...], preferred_element_type=jnp.float32)
        xw_ref[...] = xw
        xs = xw * dis
        xsa_ref[...] = xs[:, : d // 2]
        xsb_ref[...] = xs[:, d // 2:]

    return pl.pallas_call(
        body,
        grid=(10,),
        in_specs=[
            pl.BlockSpec((rows, d), lambda i: (i, 0)),
            pl.BlockSpec((d, d), lambda i: (0, 0)),
            pl.BlockSpec((rows, NW), lambda i: (i, 0)),
        ],
        out_specs=[
            pl.BlockSpec((rows, d), lambda i: (i, 0)),
            pl.BlockSpec((rows, d // 2), lambda i: (i, 0)),
            pl.BlockSpec((rows, d // 2), lambda i: (i, 0)),
        ],
        out_shape=[
            jax.ShapeDtypeStruct((n, d), jnp.float32),
            jax.ShapeDtypeStruct((n, d // 2), jnp.float32),
            jax.ShapeDtypeStruct((n, d // 2), jnp.float32),
        ],
    )(x, W, deg_t)


def _tc_mid(acc, xw_prev, deg_t, b_prev, W_next):
    n, d = acc.shape
    rows = n // 10

    def body(acc_ref, xw_ref, deg_ref, b_ref, w_ref, xwn_ref, xsa_ref, xsb_ref):
        dis = _dis_of(deg_ref)
        h = dis * acc_ref[...] + (dis * dis) * xw_ref[...] + b_ref[...]
        xwn = jnp.dot(h, w_ref[...], preferred_element_type=jnp.float32)
        xwn_ref[...] = xwn
        xs = xwn * dis
        xsa_ref[...] = xs[:, : d // 2]
        xsb_ref[...] = xs[:, d // 2:]

    return pl.pallas_call(
        body,
        grid=(10,),
        in_specs=[
            pl.BlockSpec((rows, d), lambda i: (i, 0)),
            pl.BlockSpec((rows, d), lambda i: (i, 0)),
            pl.BlockSpec((rows, NW), lambda i: (i, 0)),
            pl.BlockSpec((1, d), lambda i: (0, 0)),
            pl.BlockSpec((d, d), lambda i: (0, 0)),
        ],
        out_specs=[
            pl.BlockSpec((rows, d), lambda i: (i, 0)),
            pl.BlockSpec((rows, d // 2), lambda i: (i, 0)),
            pl.BlockSpec((rows, d // 2), lambda i: (i, 0)),
        ],
        out_shape=[
            jax.ShapeDtypeStruct((n, d), jnp.float32),
            jax.ShapeDtypeStruct((n, d // 2), jnp.float32),
            jax.ShapeDtypeStruct((n, d // 2), jnp.float32),
        ],
    )(acc, xw_prev, deg_t, b_prev, W_next)


def _tc_post(acc, xw, deg_t, b):
    n, d = acc.shape
    rows = n // 10

    def body(acc_ref, xw_ref, deg_ref, b_ref, out_ref):
        dis = _dis_of(deg_ref)
        out_ref[...] = dis * acc_ref[...] + (dis * dis) * xw_ref[...] + b_ref[...]

    return pl.pallas_call(
        body,
        grid=(10,),
        in_specs=[
            pl.BlockSpec((rows, d), lambda i: (i, 0)),
            pl.BlockSpec((rows, d), lambda i: (i, 0)),
            pl.BlockSpec((rows, NW), lambda i: (i, 0)),
            pl.BlockSpec((1, d), lambda i: (0, 0)),
        ],
        out_specs=pl.BlockSpec((rows, d), lambda i: (i, 0)),
        out_shape=jax.ShapeDtypeStruct((n, d), jnp.float32),
    )(acc, xw, deg_t, b)


def kernel(node_features, edge_indices, edge_weights, emb, W1, b1, W2, b2, W3, b3):
    G, n = node_features.shape
    d = emb.shape[1]
    np_rows = ((n + 8 * NW - 1) // (8 * NW)) * (8 * NW)

    b1r = b1.reshape(1, d)
    b2r = b2.reshape(1, d)
    b3r = b3.reshape(1, d)

    outs = []
    for g in range(G):
        nf = node_features[g]
        src = edge_indices[g, 0]
        dst = edge_indices[g, 1]
        w = edge_weights[g]

        deg32 = _degree(dst, w, np_rows)
        deg_t = deg32[:, :n].T

        x0 = _emb_gather(emb, jnp.pad(nf, (0, np_rows - n)))[:n]

        xw, xsa, xsb = _tc_pre(x0, W1, deg_t)
        for (b_r, W_next) in ((b1r, W2), (b2r, W3)):
            xws_flat = jnp.concatenate([xsa, xsb], axis=0)
            acc4 = _spmm(xws_flat, src, dst, w, np_rows, n)
            acc = jnp.concatenate([acc4[0, :n], acc4[1, :n]], axis=1)
            xw, xsa, xsb = _tc_mid(acc, xw, deg_t, b_r, W_next)

        xws_flat = jnp.concatenate([xsa, xsb], axis=0)
        acc4 = _spmm(xws_flat, src, dst, w, np_rows, n)
        acc = jnp.concatenate([acc4[0, :n], acc4[1, :n]], axis=1)
        outs.append(_tc_post(acc, xw, deg_t, b3r))

    return jnp.concatenate(outs, axis=0)
